# Initial kernel scaffold; baseline (speedup 1.0000x reference)
import jax
import jax.numpy as jnp
from jax.experimental import pallas as pl


def kernel(input_ids, word_table, pos_table, gamma, beta):
    raise NotImplementedError("write your pallas kernel here")


# speedup vs baseline: 1.9548x; 1.9548x over previous
import functools

import jax
import jax.numpy as jnp
from jax import lax
from jax.experimental import pallas as pl
from jax.experimental.pallas import tpu as pltpu
from jax.experimental.pallas import tpu_sc as plsc

NC = 2
NS = 16
NW = NC * NS
LANES = 16
CHUNK = 256


def _rsqrt16(v):
    bits = plsc.bitcast(v, jnp.int32)
    y = plsc.bitcast(jnp.int32(0x5F3759DF) - (bits >> 1), jnp.float32)
    for _ in range(3):
        y = y * (1.5 - 0.5 * v * y * y)
    return y


def _make_sc_kernel(n_tok, H, S, V):
    assert H == 8 * LANES
    assert n_tok % (NW * CHUNK) == 0
    chunks_per_w = n_tok // (NW * CHUNK)
    rows_per_chunk = CHUNK // 128

    mesh = plsc.VectorSubcoreMesh(core_axis_name="c", subcore_axis_name="s")

    @functools.partial(
        pl.kernel,
        mesh=mesh,
        out_type=jax.ShapeDtypeStruct((n_tok, H), jnp.float32),
        compiler_params=pltpu.CompilerParams(needs_layout_passes=False),
        scratch_types=[
            pltpu.VMEM((S, H), jnp.float32),
            pltpu.VMEM((H,), jnp.float32),
            pltpu.VMEM((H,), jnp.float32),
            pltpu.VMEM((2, 128), jnp.int32),
            pltpu.VMEM((CHUNK, H), jnp.float32),
            pltpu.SemaphoreType.DMA,
        ],
    )
    def body(ids2_hbm, word_hbm, pos_hbm, gamma_hbm, beta_hbm, out_hbm,
             pos_v, g_v, b_v, idx_v, rows_v, sem):
        wid = lax.axis_index("s") * NC + lax.axis_index("c")

        pltpu.sync_copy(pos_hbm.at[pl.ds(0, S)], pos_v)
        pltpu.sync_copy(gamma_hbm, g_v)
        pltpu.sync_copy(beta_hbm, b_v)

        gs = [g_v[pl.ds(j * LANES, LANES)] for j in range(8)]
        bs = [b_v[pl.ds(j * LANES, LANES)] for j in range(8)]

        def chunk_body(c, carry):
            g = wid * chunks_per_w + c
            tok_base = g * CHUNK
            pltpu.sync_copy(
                ids2_hbm.at[pl.ds(g * rows_per_chunk, rows_per_chunk)], idx_v)
            cp0 = pltpu.async_copy(
                word_hbm.at[idx_v.at[0]], rows_v.at[pl.ds(0, 128)], sem)
            cp1 = pltpu.async_copy(
                word_hbm.at[idx_v.at[1]], rows_v.at[pl.ds(128, 128)], sem)
            cp0.wait()
            cp1.wait()

            def tok_body(i, carry2):
                p = lax.rem(tok_base + i, S)
                xs = [rows_v[i, pl.ds(j * LANES, LANES)]
                      + pos_v[p, pl.ds(j * LANES, LANES)] for j in range(8)]
                s = xs[0]
                q = xs[0] * xs[0]
                for j in range(1, 8):
                    s = s + xs[j]
                    q = q + xs[j] * xs[j]
                ssum = plsc.cumsum(s)[15]
                qsum = plsc.cumsum(q)[15]
                mean = ssum * (1.0 / H)
                var = qsum * (1.0 / H) - mean * mean
                rv = _rsqrt16(jnp.broadcast_to(var + 1e-12, (LANES,)))
                for j in range(8):
                    rows_v[i, pl.ds(j * LANES, LANES)] = (
                        (xs[j] - mean) * rv * gs[j] + bs[j])
                return carry2

            lax.fori_loop(0, CHUNK, tok_body, 0)
            pltpu.sync_copy(rows_v, out_hbm.at[pl.ds(tok_base, CHUNK)])
            return carry

        lax.fori_loop(0, chunks_per_w, chunk_body, 0)

    return body


def kernel(input_ids, word_table, pos_table, gamma, beta):
    B, S = input_ids.shape
    V, H = word_table.shape
    n_tok = B * S
    ids2 = input_ids.reshape(n_tok // 128, 128)
    sc = _make_sc_kernel(n_tok, H, S, V)
    out = sc(ids2, word_table, pos_table, gamma, beta)
    return out.reshape(B, S, H)


# speedup vs baseline: 2.9154x; 1.4914x over previous
import functools

import jax
import jax.numpy as jnp
from jax import lax
from jax.experimental import pallas as pl
from jax.experimental.pallas import tpu as pltpu
from jax.experimental.pallas import tpu_sc as plsc

NC = 2
NS = 16
NW = NC * NS
LANES = 16
CHUNK = 256


def _rsqrt16(v):
    bits = plsc.bitcast(v, jnp.int32)
    y = plsc.bitcast(jnp.int32(0x5F3759DF) - (bits >> 1), jnp.float32)
    for _ in range(3):
        y = y * (1.5 - 0.5 * v * y * y)
    return y


def _make_sc_kernel(n_tok, H, S, V):
    assert H == 8 * LANES
    assert n_tok % (NW * CHUNK) == 0
    chunks_per_w = n_tok // (NW * CHUNK)
    rows_per_chunk = CHUNK // 128

    mesh = plsc.VectorSubcoreMesh(core_axis_name="c", subcore_axis_name="s")

    @functools.partial(
        pl.kernel,
        mesh=mesh,
        out_type=jax.ShapeDtypeStruct((n_tok, H), jnp.float32),
        compiler_params=pltpu.CompilerParams(needs_layout_passes=False),
        scratch_types=[
            pltpu.VMEM((S, H), jnp.float32),
            pltpu.VMEM((H,), jnp.float32),
            pltpu.VMEM((H,), jnp.float32),
            pltpu.VMEM((2, 128), jnp.int32),
            pltpu.VMEM((CHUNK, H), jnp.float32),
            pltpu.SemaphoreType.DMA,
        ],
    )
    def body(ids2_hbm, word_hbm, pos_hbm, gamma_hbm, beta_hbm, out_hbm,
             pos_v, g_v, b_v, idx_v, rows_v, sem):
        wid = lax.axis_index("s") * NC + lax.axis_index("c")

        pltpu.sync_copy(pos_hbm.at[pl.ds(0, S)], pos_v)
        pltpu.sync_copy(gamma_hbm, g_v)
        pltpu.sync_copy(beta_hbm, b_v)

        gs = [g_v[pl.ds(j * LANES, LANES)] for j in range(8)]
        bs = [b_v[pl.ds(j * LANES, LANES)] for j in range(8)]

        def chunk_body(c, carry):
            g = wid * chunks_per_w + c
            tok_base = g * CHUNK
            pltpu.sync_copy(
                ids2_hbm.at[pl.ds(g * rows_per_chunk, rows_per_chunk)], idx_v)
            cp0 = pltpu.async_copy(
                word_hbm.at[idx_v.at[0]], rows_v.at[pl.ds(0, 128)], sem)
            cp1 = pltpu.async_copy(
                word_hbm.at[idx_v.at[1]], rows_v.at[pl.ds(128, 128)], sem)
            cp0.wait()
            cp1.wait()

            @plsc.parallel_loop(0, CHUNK, 1, unroll=4)
            def tok_body(i):
                p = lax.rem(tok_base + i, S)
                xs = [rows_v[i, pl.ds(j * LANES, LANES)]
                      + pos_v[p, pl.ds(j * LANES, LANES)] for j in range(8)]
                s = xs[0]
                q = xs[0] * xs[0]
                for j in range(1, 8):
                    s = s + xs[j]
                    q = q + xs[j] * xs[j]
                ssum = plsc.cumsum(s)[15]
                qsum = plsc.cumsum(q)[15]
                mean = ssum * (1.0 / H)
                var = qsum * (1.0 / H) - mean * mean
                rv = _rsqrt16(jnp.broadcast_to(var + 1e-12, (LANES,)))
                for j in range(8):
                    rows_v[i, pl.ds(j * LANES, LANES)] = (
                        (xs[j] - mean) * rv * gs[j] + bs[j])

            pltpu.sync_copy(rows_v, out_hbm.at[pl.ds(tok_base, CHUNK)])
            return carry

        lax.fori_loop(0, chunks_per_w, chunk_body, 0)

    return body


def kernel(input_ids, word_table, pos_table, gamma, beta):
    B, S = input_ids.shape
    V, H = word_table.shape
    n_tok = B * S
    ids2 = input_ids.reshape(n_tok // 128, 128)
    sc = _make_sc_kernel(n_tok, H, S, V)
    out = sc(ids2, word_table, pos_table, gamma, beta)
    return out.reshape(B, S, H)


# speedup vs baseline: 4.0872x; 1.4019x over previous
import functools

import jax
import jax.numpy as jnp
from jax import lax
from jax.experimental import pallas as pl
from jax.experimental.pallas import tpu as pltpu
from jax.experimental.pallas import tpu_sc as plsc

NC = 2
NS = 16
NW = NC * NS
LANES = 16
CHUNK = 256


def _rsqrt16(v):
    bits = plsc.bitcast(v, jnp.int32)
    y = plsc.bitcast(jnp.int32(0x5F3759DF) - (bits >> 1), jnp.float32)
    for _ in range(3):
        y = y * (1.5 - 0.5 * v * y * y)
    return y


def _make_sc_kernel(n_tok, H, S, V):
    assert H == 8 * LANES
    assert n_tok % (NW * CHUNK) == 0
    chunks_per_w = n_tok // (NW * CHUNK)
    rows_per_chunk = CHUNK // 128

    mesh = plsc.VectorSubcoreMesh(core_axis_name="c", subcore_axis_name="s")

    @functools.partial(
        pl.kernel,
        mesh=mesh,
        out_type=jax.ShapeDtypeStruct((n_tok, H), jnp.float32),
        compiler_params=pltpu.CompilerParams(needs_layout_passes=False),
        scratch_types=[
            pltpu.VMEM((S, H), jnp.float32),
            pltpu.VMEM((H,), jnp.float32),
            pltpu.VMEM((H,), jnp.float32),
            pltpu.VMEM((2, 128), jnp.int32),
            pltpu.VMEM((CHUNK, H), jnp.float32),
            pltpu.SemaphoreType.DMA,
        ],
    )
    def body(ids2_hbm, word_hbm, pos_hbm, gamma_hbm, beta_hbm, out_hbm,
             pos_v, g_v, b_v, idx_v, rows_v, sem):
        wid = lax.axis_index("s") * NC + lax.axis_index("c")

        pltpu.sync_copy(pos_hbm.at[pl.ds(0, S)], pos_v)
        pltpu.sync_copy(gamma_hbm, g_v)
        pltpu.sync_copy(beta_hbm, b_v)

        gs = [g_v[pl.ds(j * LANES, LANES)] for j in range(8)]
        bs = [b_v[pl.ds(j * LANES, LANES)] for j in range(8)]

        def chunk_body(c, carry):
            g = wid * chunks_per_w + c
            tok_base = g * CHUNK
            pltpu.sync_copy(
                ids2_hbm.at[pl.ds(g * rows_per_chunk, rows_per_chunk)], idx_v)
            cp0 = pltpu.async_copy(
                word_hbm.at[idx_v.at[0]], rows_v.at[pl.ds(0, 128)], sem)
            cp1 = pltpu.async_copy(
                word_hbm.at[idx_v.at[1]], rows_v.at[pl.ds(128, 128)], sem)
            cp0.wait()
            cp1.wait()

            @plsc.parallel_loop(0, CHUNK, 1, unroll=2)
            def tok_body(i):
                p = lax.rem(tok_base + i, S)
                xs = [rows_v[i, pl.ds(j * LANES, LANES)]
                      + pos_v[p, pl.ds(j * LANES, LANES)] for j in range(8)]
                s = xs[0]
                q = xs[0] * xs[0]
                for j in range(1, 8):
                    s = s + xs[j]
                    q = q + xs[j] * xs[j]
                ssum = plsc.cumsum(s)[15]
                qsum = plsc.cumsum(q)[15]
                mean = ssum * (1.0 / H)
                var = qsum * (1.0 / H) - mean * mean
                rv = _rsqrt16(jnp.broadcast_to(var + 1e-12, (LANES,)))
                for j in range(8):
                    rows_v[i, pl.ds(j * LANES, LANES)] = (
                        (xs[j] - mean) * rv * gs[j] + bs[j])

            pltpu.sync_copy(rows_v, out_hbm.at[pl.ds(tok_base, CHUNK)])
            return carry

        lax.fori_loop(0, chunks_per_w, chunk_body, 0)

    return body


def kernel(input_ids, word_table, pos_table, gamma, beta):
    B, S = input_ids.shape
    V, H = word_table.shape
    n_tok = B * S
    ids2 = input_ids.reshape(n_tok // 128, 128)
    sc = _make_sc_kernel(n_tok, H, S, V)
    out = sc(ids2, word_table, pos_table, gamma, beta)
    return out.reshape(B, S, H)
